# baseline (speedup 1.0000x reference)
import jax
import jax.numpy as jnp
from jax import lax
from jax.experimental import pallas as pl
from jax.experimental.pallas import tpu as pltpu
from jax.experimental.pallas import tpu_sc as plsc

NUM_QUERY = 4096
NUM_NEG = 128
D = 64
L = 16
GAMMA = 12.0
NW = 32
QPW = NUM_QUERY // NW


def _sc_body(x_hbm, emb_hbm, tgt_hbm, posidx_hbm, negidx_hbm,
             pos_out_hbm, neg_out_hbm,
             tgt_v, posidx_v, negidx_v, pred_v, pos_v, nbuf0, nbuf1,
             t_v, tpos_v, pos_out_v, neg_out_v,
             sem_pred, sem_pos, sem_n0, sem_n1):
    cid = lax.axis_index("c")
    sid = lax.axis_index("s")
    wid = sid * 2 + cid
    base = wid * QPW

    pltpu.sync_copy(tgt_hbm.at[pl.ds(base, QPW)], tgt_v)
    pltpu.sync_copy(posidx_hbm.at[pl.ds(base, QPW)], posidx_v)
    pltpu.sync_copy(negidx_hbm.at[pl.ds(base, QPW)], negidx_v)

    cp_pred = pltpu.async_copy(x_hbm.at[tgt_v], pred_v, sem_pred)
    cp_pos = pltpu.async_copy(emb_hbm.at[posidx_v], pos_v, sem_pos)
    pltpu.async_copy(emb_hbm.at[negidx_v.at[0]], nbuf0, sem_n0)
    cp_pred.wait()
    cp_pos.wait()

    def wait_nbuf(nbuf, sem):
        pltpu.make_async_copy(emb_hbm.at[pl.ds(0, NUM_NEG)], nbuf, sem).wait()

    iota = lax.iota(jnp.int32, L)

    def lane_reduce(tref, g):
        rows = g * L + iota
        acc = plsc.load_gather(tref, [rows, jnp.zeros((L,), jnp.int32)])
        for c in range(1, L):
            acc = acc + plsc.load_gather(
                tref, [rows, jnp.full((L,), c, jnp.int32)])
        return acc

    def compute(q, nbuf):
        p0 = pred_v[q, pl.ds(0, L)]
        p1 = pred_v[q, pl.ds(L, L)]
        p2 = pred_v[q, pl.ds(2 * L, L)]
        p3 = pred_v[q, pl.ds(3 * L, L)]

        a = jnp.abs(pos_v[q, pl.ds(0, L)] - p0)
        a = a + jnp.abs(pos_v[q, pl.ds(L, L)] - p1)
        a = a + jnp.abs(pos_v[q, pl.ds(2 * L, L)] - p2)
        a = a + jnp.abs(pos_v[q, pl.ds(3 * L, L)] - p3)
        tpos_v[q, pl.ds(0, L)] = a

        @pl.loop(0, NUM_NEG, unroll=4)
        def _neg(j):
            b = jnp.abs(nbuf[j, pl.ds(0, L)] - p0)
            b = b + jnp.abs(nbuf[j, pl.ds(L, L)] - p1)
            b = b + jnp.abs(nbuf[j, pl.ds(2 * L, L)] - p2)
            b = b + jnp.abs(nbuf[j, pl.ds(3 * L, L)] - p3)
            t_v[j, pl.ds(0, L)] = b

        for g in range(NUM_NEG // L):
            neg_out_v[q, pl.ds(g * L, L)] = GAMMA - lane_reduce(t_v, g)

    @pl.loop(0, QPW, step=2)
    def _q(q):
        pltpu.async_copy(emb_hbm.at[negidx_v.at[q + 1]], nbuf1, sem_n1)
        wait_nbuf(nbuf0, sem_n0)
        compute(q, nbuf0)

        @pl.when(q + 2 < QPW)
        def _():
            pltpu.async_copy(emb_hbm.at[negidx_v.at[q + 2]], nbuf0, sem_n0)

        wait_nbuf(nbuf1, sem_n1)
        compute(q + 1, nbuf1)

    for g in range(QPW // L):
        pos_out_v[pl.ds(g * L, L)] = GAMMA - lane_reduce(tpos_v, g)

    pltpu.sync_copy(pos_out_v, pos_out_hbm.at[pl.ds(base, QPW)])
    pltpu.sync_copy(neg_out_v, neg_out_hbm.at[pl.ds(base, QPW)])


@jax.jit
def _run(x, emb, tgt, pos, neg):
    mesh = plsc.VectorSubcoreMesh(core_axis_name="c", subcore_axis_name="s")
    f = pl.kernel(
        _sc_body,
        out_type=(
            jax.ShapeDtypeStruct((NUM_QUERY,), jnp.float32),
            jax.ShapeDtypeStruct((NUM_QUERY, NUM_NEG), jnp.float32),
        ),
        mesh=mesh,
        compiler_params=pltpu.CompilerParams(
            needs_layout_passes=False, use_tc_tiling_on_sc=False),
        scratch_types=[
            pltpu.VMEM((QPW,), jnp.int32),
            pltpu.VMEM((QPW,), jnp.int32),
            pltpu.VMEM((QPW, NUM_NEG), jnp.int32),
            pltpu.VMEM((QPW, D), jnp.float32),
            pltpu.VMEM((QPW, D), jnp.float32),
            pltpu.VMEM((NUM_NEG, D), jnp.float32),
            pltpu.VMEM((NUM_NEG, D), jnp.float32),
            pltpu.VMEM((NUM_NEG, L + 1), jnp.float32),
            pltpu.VMEM((QPW, L + 1), jnp.float32),
            pltpu.VMEM((QPW,), jnp.float32),
            pltpu.VMEM((QPW, NUM_NEG), jnp.float32),
            pltpu.SemaphoreType.DMA,
            pltpu.SemaphoreType.DMA,
            pltpu.SemaphoreType.DMA,
            pltpu.SemaphoreType.DMA,
        ],
    )
    return f(x, emb, tgt, pos, neg)


def kernel(x, entity_embedding, target_node_idxes, positive_samples,
           negative_samples):
    tgt = target_node_idxes.astype(jnp.int32)
    pos = positive_samples.astype(jnp.int32)
    neg = negative_samples.astype(jnp.int32)
    pos_logit, neg_logit = _run(x, entity_embedding, tgt, pos, neg)
    return (pos_logit[:, None], neg_logit)
